# Initial kernel scaffold; baseline (speedup 1.0000x reference)
import jax
import jax.numpy as jnp
from jax.experimental import pallas as pl


def kernel(tables, feat_0, feat_1, feat_2, feat_3):
    raise NotImplementedError("write your pallas kernel here")


# speedup vs baseline: 1.1523x; 1.1523x over previous
import functools

import jax
import jax.numpy as jnp
from jax import lax
from jax.experimental import pallas as pl
from jax.experimental.pallas import tpu as pltpu
from jax.experimental.pallas import tpu_sc as plsc

NUM_FEATURES = 4
CHUNKS_PER_FEATURE = 2
NUM_TABLES = 8
BUCKETS = 100000
DIM = 32
BATCH = 16384

NUM_WORKERS = 32
WORKERS_PER_CHUNK = NUM_WORKERS // NUM_TABLES
ROWS_PER_WORKER = BATCH // WORKERS_PER_CHUNK
BLK = 1024
NBLK = ROWS_PER_WORKER // BLK
SUB = 128
NSUB = BLK // SUB
LANES = 16


def _body(tab_hbm, feats_hbm, out_hbm, feat_v, idx_v, rows_v, sem):
    wid = lax.axis_index("s") * 2 + lax.axis_index("c")
    chunk = wid // WORKERS_PER_CHUNK
    quarter = wid % WORKERS_PER_CHUNK
    f = chunk // CHUNKS_PER_FEATURE
    c = chunk % CHUNKS_PER_FEATURE

    f_u = f.astype(jnp.uint32)
    c_u = c.astype(jnp.uint32)
    mult0 = jnp.uint32(2654435761) + jnp.uint32(2) * f_u + jnp.uint32(1)
    add0 = c_u * jnp.uint32(40503) + jnp.uint32(97)
    tab_base = chunk.astype(jnp.uint32) * jnp.uint32(BUCKETS)

    feat_base = f * BATCH + quarter * ROWS_PER_WORKER
    out_base = feat_base
    col0 = c * DIM

    def do_block(blk, _):
        row0 = blk * BLK
        pltpu.sync_copy(feats_hbm.at[pl.dslice(feat_base + row0, BLK)], feat_v)

        def hash_row(j, _):
            for ii in range(SUB // LANES):
                x = feat_v[pl.dslice(j * SUB + ii * LANES, LANES)]
                h = x.astype(jnp.uint32)
                h = h * mult0
                h = h + add0
                h = h ^ (h >> jnp.uint32(16))
                h = h * jnp.uint32(2246822519)
                h = h ^ (h >> jnp.uint32(13))
                h = h % jnp.uint32(BUCKETS)
                h = h + tab_base
                idx_v[j, pl.dslice(ii * LANES, LANES)] = h.astype(jnp.int32)
            return 0

        lax.fori_loop(0, NSUB, hash_row, 0)

        copies = [
            pltpu.async_copy(
                tab_hbm.at[idx_v.at[j]],
                rows_v.at[pl.dslice(j * SUB, SUB)],
                sem,
            )
            for j in range(NSUB)
        ]
        for cp in copies:
            cp.wait()

        pltpu.sync_copy(
            rows_v,
            out_hbm.at[pl.dslice(out_base + row0, BLK), pl.dslice(col0, DIM)],
        )
        return 0

    lax.fori_loop(0, NBLK, do_block, 0)


def kernel(tables, feat_0, feat_1, feat_2, feat_3):
    tab2d = tables.reshape(NUM_TABLES * BUCKETS, DIM)
    feats = jnp.stack([feat_0, feat_1, feat_2, feat_3]).reshape(NUM_FEATURES * BATCH)

    mesh = plsc.VectorSubcoreMesh(core_axis_name="c", subcore_axis_name="s")
    run = functools.partial(
        pl.kernel,
        out_type=jax.ShapeDtypeStruct((NUM_FEATURES * BATCH, CHUNKS_PER_FEATURE * DIM),
                                      jnp.float32),
        mesh=mesh,
        scratch_types=[
            pltpu.VMEM((BLK,), jnp.int32),
            pltpu.VMEM((NSUB, SUB), jnp.int32),
            pltpu.VMEM((BLK, DIM), jnp.float32),
            pltpu.SemaphoreType.DMA,
        ],
        compiler_params=pltpu.CompilerParams(use_tc_tiling_on_sc=False),
    )(_body)

    out2d = run(tab2d, feats)
    return out2d.reshape(NUM_FEATURES, BATCH, CHUNKS_PER_FEATURE * DIM)


# speedup vs baseline: 1.1549x; 1.0022x over previous
import functools

import jax
import jax.numpy as jnp
from jax import lax
from jax.experimental import pallas as pl
from jax.experimental.pallas import tpu as pltpu
from jax.experimental.pallas import tpu_sc as plsc

NUM_FEATURES = 4
CHUNKS_PER_FEATURE = 2
NUM_TABLES = 8
BUCKETS = 100000
DIM = 32
BATCH = 16384

NUM_WORKERS = 32
WORKERS_PER_CHUNK = NUM_WORKERS // NUM_TABLES
ROWS_PER_WORKER = BATCH // WORKERS_PER_CHUNK
BLK = 1024
NBLK = ROWS_PER_WORKER // BLK
SUB = 128
NSUB = BLK // SUB
LANES = 16


def _body(tab_hbm, feats_hbm, out_hbm, feat_v, idx_v, rows_v, sem):
    wid = lax.axis_index("s") * 2 + lax.axis_index("c")
    chunk = wid // WORKERS_PER_CHUNK
    quarter = wid % WORKERS_PER_CHUNK
    f = chunk // CHUNKS_PER_FEATURE
    c = chunk % CHUNKS_PER_FEATURE

    f_u = f.astype(jnp.uint32)
    c_u = c.astype(jnp.uint32)
    mult0 = jnp.uint32(2654435761) + jnp.uint32(2) * f_u + jnp.uint32(1)
    add0 = c_u * jnp.uint32(40503) + jnp.uint32(97)

    row_base = quarter * ROWS_PER_WORKER
    feat_base = f * BATCH + row_base
    col0 = c * DIM

    def do_block(blk, _):
        row0 = blk * BLK
        pltpu.sync_copy(feats_hbm.at[pl.dslice(feat_base + row0, BLK)], feat_v)

        def hash_row(j, _):
            for ii in range(SUB // LANES):
                x = feat_v[pl.dslice(j * SUB + ii * LANES, LANES)]
                h = x.astype(jnp.uint32)
                h = h * mult0
                h = h + add0
                h = h ^ (h >> jnp.uint32(16))
                h = h * jnp.uint32(2246822519)
                h = h ^ (h >> jnp.uint32(13))
                h = h % jnp.uint32(BUCKETS)
                idx_v[j, pl.dslice(ii * LANES, LANES)] = h.astype(jnp.int32)
            return 0

        lax.fori_loop(0, NSUB, hash_row, 0)

        copies = [
            pltpu.async_copy(
                tab_hbm.at[chunk].at[idx_v.at[j]],
                rows_v.at[pl.dslice(j * SUB, SUB)],
                sem,
            )
            for j in range(NSUB)
        ]
        for cp in copies:
            cp.wait()

        pltpu.sync_copy(
            rows_v,
            out_hbm.at[f, pl.dslice(row_base + row0, BLK), pl.dslice(col0, DIM)],
        )
        return 0

    lax.fori_loop(0, NBLK, do_block, 0)


def kernel(tables, feat_0, feat_1, feat_2, feat_3):
    feats = jnp.stack([feat_0, feat_1, feat_2, feat_3]).reshape(NUM_FEATURES * BATCH)

    mesh = plsc.VectorSubcoreMesh(core_axis_name="c", subcore_axis_name="s")
    run = functools.partial(
        pl.kernel,
        out_type=jax.ShapeDtypeStruct(
            (NUM_FEATURES, BATCH, CHUNKS_PER_FEATURE * DIM), jnp.float32),
        mesh=mesh,
        scratch_types=[
            pltpu.VMEM((BLK,), jnp.int32),
            pltpu.VMEM((NSUB, SUB), jnp.int32),
            pltpu.VMEM((BLK, DIM), jnp.float32),
            pltpu.SemaphoreType.DMA,
        ],
        compiler_params=pltpu.CompilerParams(use_tc_tiling_on_sc=False),
    )(_body)

    return run(tables, feats)
